# baseline (speedup 1.0000x reference)
import jax
import jax.numpy as jnp
from jax.experimental import pallas as pl

N = 32768
D = 256
H = 128
M = 16
BLK = 4096
NSTREAM = 2
G = N // (BLK * NSTREAM)


def _fused_body(*refs):
    x_refs = refs[:NSTREAM]
    idx_refs = refs[NSTREAM:2 * NSTREAM]
    w1_ref, b1_ref, w2_ref, b2_ref, out_ref = refs[2 * NSTREAM:]
    i = pl.program_id(0)

    @pl.when(i == 0)
    def _init():
        out_ref[...] = jnp.zeros_like(out_ref)

    w1 = w1_ref[...].astype(jnp.bfloat16)
    acc = jnp.zeros((1, M), dtype=jnp.float32)
    for s in range(NSTREAM):
        x = x_refs[s][...].astype(jnp.bfloat16)
        h = jnp.dot(x, w1, preferred_element_type=jnp.float32)
        h = h + b1_ref[...]
        h = h * jax.nn.sigmoid(h)
        y = jnp.dot(h, w2_ref[...], preferred_element_type=jnp.float32)
        y = y + b2_ref[...]
        idx = idx_refs[s][...]
        sel = (idx == jax.lax.broadcasted_iota(jnp.int32, (1, M), 1)).astype(
            jnp.float32)
        acc = acc + jax.lax.dot_general(
            y, sel, (((0,), (0,)), ((), ())),
            preferred_element_type=jnp.float32)
    out_ref[...] += acc


def kernel(scalar_representation, idx_m, W1, b1, W2, b2):
    idx2d = idx_m.astype(jnp.int32).reshape(N, 1)

    def x_spec(s):
        return pl.BlockSpec((BLK, D), lambda i, s=s: (s * G + i, 0))

    def idx_spec(s):
        return pl.BlockSpec((BLK, 1), lambda i, s=s: (s * G + i, 0))

    in_specs = ([x_spec(s) for s in range(NSTREAM)]
                + [idx_spec(s) for s in range(NSTREAM)]
                + [pl.BlockSpec((D, H), lambda i: (0, 0)),
                   pl.BlockSpec((1, H), lambda i: (0, 0)),
                   pl.BlockSpec((H, 1), lambda i: (0, 0)),
                   pl.BlockSpec((1, 1), lambda i: (0, 0))])
    args = ([scalar_representation] * NSTREAM + [idx2d] * NSTREAM
            + [W1, b1.reshape(1, H), W2, b2.reshape(1, 1)])
    out = pl.pallas_call(
        _fused_body,
        grid=(G,),
        in_specs=in_specs,
        out_specs=pl.BlockSpec((1, M), lambda i: (0, 0)),
        out_shape=jax.ShapeDtypeStruct((1, M), jnp.float32),
    )(*args)
    return out.reshape(M)
